# baseline (speedup 1.0000x reference)
import functools

import jax
import jax.numpy as jnp
from jax.experimental import pallas as pl
from jax.experimental.pallas import tpu as pltpu

BS = 128
D1, D2, D3 = 256, 512, 100000
BLK = 2048
NBLK = (D3 + BLK - 1) // BLK
OH_BLK = 4096
NEG = -1e30
K = 5
CARRY = 128


def _leaky(x):
    return jnp.where(x >= 0, x, 0.2 * x)


def _topk_body(z_ref, w1_ref, b1_ref, w2_ref, b2_ref, g2_ref, be2_ref,
               w3_ref, b3_ref, g3_ref, be3_ref,
               idx_out_ref, h2_ref, cv_ref, ci_ref):
    j = pl.program_id(0)

    @pl.when(j == 0)
    def _prologue():
        h1 = _leaky(jnp.dot(z_ref[...], w1_ref[...],
                            preferred_element_type=jnp.float32) + b1_ref[...])
        t = jnp.dot(h1, w2_ref[...],
                    preferred_element_type=jnp.float32) + b2_ref[...]
        mean = jnp.mean(t, axis=0, keepdims=True)
        var = jnp.mean((t - mean) ** 2, axis=0, keepdims=True)
        h2 = (t - mean) * jax.lax.rsqrt(var + 0.8) * g2_ref[...] + be2_ref[...]
        h2_ref[...] = _leaky(h2)
        cv_ref[...] = jnp.full((BS, CARRY), NEG, dtype=jnp.float32)
        ci_ref[...] = jnp.zeros((BS, CARRY), dtype=jnp.int32)

    x = jnp.dot(h2_ref[...], w3_ref[...],
                preferred_element_type=jnp.float32) + b3_ref[...]
    mean3 = jnp.mean(x, axis=0, keepdims=True)
    var3 = jnp.mean((x - mean3) ** 2, axis=0, keepdims=True)
    xn = (x - mean3) * jax.lax.rsqrt(var3 + 0.8) * g3_ref[...] + be3_ref[...]

    base = j * BLK
    cols = base + jax.lax.broadcasted_iota(jnp.int32, (BS, BLK), 1)
    xn = jnp.where(cols < D3, xn, NEG)

    cv = cv_ref[...]
    ci = ci_ref[...]
    x_ext = jnp.concatenate([cv, xn], axis=1)
    lane_ext = jax.lax.broadcasted_iota(jnp.int32, (BS, CARRY + BLK), 1)
    lane_c = jax.lax.broadcasted_iota(jnp.int32, (BS, CARRY), 1)

    newv = jnp.full((BS, CARRY), NEG, dtype=jnp.float32)
    newi = jnp.zeros((BS, CARRY), dtype=jnp.int32)
    for t in range(K):
        m = jnp.max(x_ext, axis=1)
        am = jnp.min(jnp.where(x_ext == m[:, None], lane_ext, CARRY + BLK),
                     axis=1)
        carry_g = jnp.sum(jnp.where(lane_c == am[:, None], ci, 0), axis=1)
        gidx = jnp.where(am < CARRY, carry_g, base + am - CARRY)
        newv = jnp.where(lane_c == t, m[:, None], newv)
        newi = jnp.where(lane_c == t, gidx[:, None], newi)
        x_ext = jnp.where(lane_ext == am[:, None], NEG, x_ext)

    cv_ref[...] = newv
    ci_ref[...] = newi
    idx_out_ref[...] = newi


def _onehot_body(idx_ref, out_ref):
    j = pl.program_id(0)
    cols = j * OH_BLK + jax.lax.broadcasted_iota(jnp.int32, (BS * K, OH_BLK), 1)
    out_ref[...] = jnp.where(cols == idx_ref[...], 1.0, 0.0).astype(jnp.float32)


@functools.partial(jax.jit, static_argnums=(0,))
def _run(bs_static, z, W1, b1, W2, b2, gamma2, beta2, W3, b3, gamma3, beta3):
    full = lambda shape: pl.BlockSpec(shape, lambda j: (0, 0))
    colblk = lambda r: pl.BlockSpec((r, BLK), lambda j: (0, j))

    idx_pad = pl.pallas_call(
        _topk_body,
        grid=(NBLK,),
        in_specs=[
            full((BS, BS)),
            full((BS, D1)),
            full((1, D1)),
            full((D1, D2)),
            full((1, D2)),
            full((1, D2)),
            full((1, D2)),
            colblk(D2),
            colblk(1),
            colblk(1),
            colblk(1),
        ],
        out_specs=pl.BlockSpec((BS, CARRY), lambda j: (0, 0)),
        out_shape=jax.ShapeDtypeStruct((BS, CARRY), jnp.int32),
        scratch_shapes=[
            pltpu.VMEM((BS, D2), jnp.float32),
            pltpu.VMEM((BS, CARRY), jnp.float32),
            pltpu.VMEM((BS, CARRY), jnp.int32),
        ],
    )(z, W1, b1.reshape(1, D1), W2, b2.reshape(1, D2),
      gamma2.reshape(1, D2), beta2.reshape(1, D2),
      W3, b3.reshape(1, D3), gamma3.reshape(1, D3), beta3.reshape(1, D3))

    idx = idx_pad[:, :K]
    idx_flat = idx.reshape(BS * K, 1)

    oh2d = pl.pallas_call(
        _onehot_body,
        grid=((D3 + OH_BLK - 1) // OH_BLK,),
        in_specs=[pl.BlockSpec((BS * K, 1), lambda j: (0, 0))],
        out_specs=pl.BlockSpec((BS * K, OH_BLK), lambda j: (0, j)),
        out_shape=jax.ShapeDtypeStruct((BS * K, D3), jnp.float32),
    )(idx_flat)

    return oh2d.reshape(BS, K, D3), idx


def kernel(bs, z, W1, b1, W2, b2, gamma2, beta2, W3, b3, gamma3, beta3):
    return _run(z.shape[0], z, W1, b1, W2, b2, gamma2, beta2,
                W3, b3, gamma3, beta3)


# speedup vs baseline: 1.1374x; 1.1374x over previous
import functools

import jax
import jax.numpy as jnp
from jax.experimental import pallas as pl
from jax.experimental.pallas import tpu as pltpu

BS = 128
D1, D2, D3 = 256, 512, 100000
BLK = 2048
NBLK = (D3 + BLK - 1) // BLK
OH_BLK = 4096
NEG = -1e30
K = 5
CARRY = 128


def _leaky(x):
    return jnp.where(x >= 0, x, 0.2 * x)


def _topk_body(z_ref, w1_ref, b1_ref, w2_ref, b2_ref, g2_ref, be2_ref,
               w3_ref, b3_ref, g3_ref, be3_ref,
               idx_out_ref, h2_ref, cv_ref, ci_ref):
    j = pl.program_id(0)

    @pl.when(j == 0)
    def _prologue():
        h1 = _leaky(jnp.dot(z_ref[...], w1_ref[...],
                            preferred_element_type=jnp.float32) + b1_ref[...])
        t = jnp.dot(h1, w2_ref[...],
                    preferred_element_type=jnp.float32) + b2_ref[...]
        mean = jnp.mean(t, axis=0, keepdims=True)
        var = jnp.mean((t - mean) ** 2, axis=0, keepdims=True)
        h2 = (t - mean) * jax.lax.rsqrt(var + 0.8) * g2_ref[...] + be2_ref[...]
        h2_ref[...] = _leaky(h2)
        cv_ref[...] = jnp.full((BS, CARRY), NEG, dtype=jnp.float32)
        ci_ref[...] = jnp.zeros((BS, CARRY), dtype=jnp.int32)

    x = jnp.dot(h2_ref[...], w3_ref[...],
                preferred_element_type=jnp.float32) + b3_ref[...]
    mean3 = jnp.mean(x, axis=0, keepdims=True)
    var3 = jnp.mean((x - mean3) ** 2, axis=0, keepdims=True)
    xn = (x - mean3) * jax.lax.rsqrt(var3 + 0.8) * g3_ref[...] + be3_ref[...]

    base = j * BLK
    cols = base + jax.lax.broadcasted_iota(jnp.int32, (BS, BLK), 1)
    xn = jnp.where(cols < D3, xn, NEG)

    cv = cv_ref[...]
    ci = ci_ref[...]
    x_ext = jnp.concatenate([cv, xn], axis=1)
    lane_ext = jax.lax.broadcasted_iota(jnp.int32, (BS, CARRY + BLK), 1)
    lane_c = jax.lax.broadcasted_iota(jnp.int32, (BS, CARRY), 1)

    newv = jnp.full((BS, CARRY), NEG, dtype=jnp.float32)
    newi = jnp.zeros((BS, CARRY), dtype=jnp.int32)
    for t in range(K):
        m = jnp.max(x_ext, axis=1)
        am = jnp.min(jnp.where(x_ext == m[:, None], lane_ext, CARRY + BLK),
                     axis=1)
        carry_g = jnp.sum(jnp.where(lane_c == am[:, None], ci, 0), axis=1)
        gidx = jnp.where(am < CARRY, carry_g, base + am - CARRY)
        newv = jnp.where(lane_c == t, m[:, None], newv)
        newi = jnp.where(lane_c == t, gidx[:, None], newi)
        x_ext = jnp.where(lane_ext == am[:, None], NEG, x_ext)

    cv_ref[...] = newv
    ci_ref[...] = newi
    idx_out_ref[...] = newi


def _onehot_body(idx_ref, out_ref):
    j = pl.program_id(0)
    cols = j * OH_BLK + jax.lax.broadcasted_iota(jnp.int32, (BS, K, OH_BLK), 2)
    out_ref[...] = jnp.where(cols == idx_ref[...][:, :, None],
                             1.0, 0.0).astype(jnp.float32)


@functools.partial(jax.jit, static_argnums=(0,))
def _run(bs_static, z, W1, b1, W2, b2, gamma2, beta2, W3, b3, gamma3, beta3):
    full = lambda shape: pl.BlockSpec(shape, lambda j: (0, 0))
    colblk = lambda r: pl.BlockSpec((r, BLK), lambda j: (0, j))

    idx_pad = pl.pallas_call(
        _topk_body,
        grid=(NBLK,),
        in_specs=[
            full((BS, BS)),
            full((BS, D1)),
            full((1, D1)),
            full((D1, D2)),
            full((1, D2)),
            full((1, D2)),
            full((1, D2)),
            colblk(D2),
            colblk(1),
            colblk(1),
            colblk(1),
        ],
        out_specs=pl.BlockSpec((BS, CARRY), lambda j: (0, 0)),
        out_shape=jax.ShapeDtypeStruct((BS, CARRY), jnp.int32),
        scratch_shapes=[
            pltpu.VMEM((BS, D2), jnp.float32),
            pltpu.VMEM((BS, CARRY), jnp.float32),
            pltpu.VMEM((BS, CARRY), jnp.int32),
        ],
    )(z, W1, b1.reshape(1, D1), W2, b2.reshape(1, D2),
      gamma2.reshape(1, D2), beta2.reshape(1, D2),
      W3, b3.reshape(1, D3), gamma3.reshape(1, D3), beta3.reshape(1, D3))

    idx = idx_pad[:, :K]

    oh = pl.pallas_call(
        _onehot_body,
        grid=((D3 + OH_BLK - 1) // OH_BLK,),
        in_specs=[pl.BlockSpec((BS, K), lambda j: (0, 0))],
        out_specs=pl.BlockSpec((BS, K, OH_BLK), lambda j: (0, 0, j)),
        out_shape=jax.ShapeDtypeStruct((BS, K, D3), jnp.float32),
    )(idx)

    return oh, idx


def kernel(bs, z, W1, b1, W2, b2, gamma2, beta2, W3, b3, gamma3, beta3):
    return _run(z.shape[0], z, W1, b1, W2, b2, gamma2, beta2,
                W3, b3, gamma3, beta3)
